# Initial kernel scaffold; baseline (speedup 1.0000x reference)
import jax
import jax.numpy as jnp
from jax.experimental import pallas as pl


def kernel(pos, Z, atom_types, edge_index):
    raise NotImplementedError("write your pallas kernel here")


# speedup vs baseline: 89.7446x; 89.7446x over previous
import functools

import jax
import jax.numpy as jnp
from jax import lax
from jax.experimental import pallas as pl
from jax.experimental.pallas import tpu as pltpu
from jax.experimental.pallas import tpu_sc as plsc

R_MAX = 5.0
QQ = 14.399645 * 0.5
PZBL = 0.23
A0 = 0.4685
C1, C2, C3, C4 = 0.02817, 0.28022, 0.50986, 0.18175
D1, D2, D3, D4 = -0.20162, -0.4029, -0.94229, -3.1998

NC = 2
NS = 16
NW = NC * NS
L = 16

CHUNK = 512
IPD = 128
MSEG = 16


def _rsqrt(r2):
    bits = plsc.bitcast(r2, jnp.int32)
    y = plsc.bitcast(jnp.int32(0x5F3759DF) - (bits >> 1), jnp.float32)
    for _ in range(3):
        y = y * (1.5 - 0.5 * r2 * y * y)
    return y


def _edge_energy(sx, sy, sz, szp, szq, dx, dy, dz, dzp, dzq):
    ex = sx - dx
    ey = sy - dy
    ez = sz - dz
    r2 = ex * ex + ey * ey + ez * ez
    rinv = _rsqrt(r2)
    r = r2 * rinv
    x = (szp + dzp) * (r * (1.0 / A0))
    psi = (C1 * jnp.exp(D1 * x) + C2 * jnp.exp(D2 * x)
           + C3 * jnp.exp(D3 * x) + C4 * jnp.exp(D4 * x))
    eng = (szq * dzq) * rinv * psi
    rn = r * (1.0 / R_MAX)
    rn2 = rn * rn
    rn4 = rn2 * rn2
    rn6 = rn4 * rn2
    rn7 = rn6 * rn
    rn8 = rn7 * rn
    cut = 1.0 - 28.0 * rn6 + 48.0 * rn7 - 21.0 * rn8
    return jnp.where(rn < 1.0, cut * eng, 0.0)


def _make_sc_kernel(n_pad, n_edges):
    n_chunks = n_edges // CHUNK
    base_chunks = n_chunks // NW
    extra = n_chunks - base_chunks * NW
    seg_len = n_pad // MSEG
    sl = seg_len // NS
    vecs = CHUNK // L
    ndesc = CHUNK // IPD

    mesh = plsc.VectorSubcoreMesh(
        core_axis_name="c", subcore_axis_name="s", num_cores=NC,
        num_subcores=NS)

    @functools.partial(
        pl.kernel,
        out_type=jax.ShapeDtypeStruct((NC * n_pad,), jnp.float32),
        mesh=mesh,
        scratch_types=[
            pltpu.VMEM((n_pad,), jnp.float32),
            pltpu.VMEM((2 * CHUNK,), jnp.int32),
            pltpu.VMEM((CHUNK, 8), jnp.float32),
            pltpu.VMEM((CHUNK, 8), jnp.float32),
            pltpu.VMEM((sl,), jnp.float32),
            pltpu.VMEM((sl,), jnp.float32),
            pltpu.VMEM_SHARED((NS * seg_len,), jnp.float32),
            pltpu.SemaphoreType.DMA,
        ],
        compiler_params=pltpu.CompilerParams(
            needs_layout_passes=False, use_tc_tiling_on_sc=False),
    )
    def zbl(table_hbm, edges_hbm, out_hbm, acc, eidx, srows, drows,
            msum, mtmp, shared, sem):
        cid = lax.axis_index("c")
        sid = lax.axis_index("s")
        wid = cid * NS + sid
        lane = lax.iota(jnp.int32, 16)
        zero_l = lane * 0

        def zero_body(i, _):
            acc[pl.ds(i * L, L)] = zero_l.astype(jnp.float32) * 0.0
            return 0

        lax.fori_loop(0, n_pad // L, zero_body, 0)

        n_my = base_chunks + jnp.where(wid < extra, 1, 0)

        def chunk_body(i, _):
            base = (wid + i * NW) * CHUNK
            pltpu.sync_copy(edges_hbm.at[pl.ds(base, CHUNK)],
                            eidx.at[pl.ds(0, CHUNK)])
            pltpu.sync_copy(edges_hbm.at[pl.ds(n_edges + base, CHUNK)],
                            eidx.at[pl.ds(CHUNK, CHUNK)])
            cps = []
            for j in range(ndesc):
                cps.append(pltpu.async_copy(
                    table_hbm.at[eidx.at[pl.ds(j * IPD, IPD)]],
                    srows.at[pl.ds(j * IPD, IPD)], sem))
                cps.append(pltpu.async_copy(
                    table_hbm.at[eidx.at[pl.ds(CHUNK + j * IPD, IPD)]],
                    drows.at[pl.ds(j * IPD, IPD)], sem))
            for cp in cps:
                cp.wait()

            def vec_body(v, _):
                rid = lane + v * L
                sxv = plsc.load_gather(srows, [rid, zero_l])
                syv = plsc.load_gather(srows, [rid, zero_l + 1])
                szv = plsc.load_gather(srows, [rid, zero_l + 2])
                szp = plsc.load_gather(srows, [rid, zero_l + 3])
                szq = plsc.load_gather(srows, [rid, zero_l + 4])
                dxv = plsc.load_gather(drows, [rid, zero_l])
                dyv = plsc.load_gather(drows, [rid, zero_l + 1])
                dzv = plsc.load_gather(drows, [rid, zero_l + 2])
                dzp = plsc.load_gather(drows, [rid, zero_l + 3])
                dzq = plsc.load_gather(drows, [rid, zero_l + 4])
                e = _edge_energy(sxv, syv, szv, szp, szq,
                                 dxv, dyv, dzv, dzp, dzq)
                sidx = eidx[pl.ds(v * L, L)]
                plsc.addupdate_scatter(acc, [sidx], e)
                return 0

            lax.fori_loop(0, vecs, vec_body, 0)
            return 0

        lax.fori_loop(0, n_my, chunk_body, 0)

        mbase = sid * sl
        for g in range(MSEG):
            seg0 = g * seg_len
            pltpu.sync_copy(acc.at[pl.ds(seg0, seg_len)],
                            shared.at[pl.ds(sid * seg_len, seg_len)])
            plsc.subcore_barrier()
            pltpu.sync_copy(shared.at[pl.ds(mbase, sl)], msum)

            def merge_tile(t, _):
                pltpu.sync_copy(shared.at[pl.ds(t * seg_len + mbase, sl)],
                                mtmp)

                def add_body(k, _):
                    msum[pl.ds(k * L, L)] = msum[pl.ds(k * L, L)] + \
                        mtmp[pl.ds(k * L, L)]
                    return 0

                lax.fori_loop(0, sl // L, add_body, 0)
                return 0

            lax.fori_loop(1, NS, merge_tile, 0)
            pltpu.sync_copy(
                msum, out_hbm.at[pl.ds(cid * n_pad + seg0 + mbase, sl)])
            plsc.subcore_barrier()

    return zbl


def kernel(pos, Z, atom_types, edge_index):
    n = pos.shape[0]
    n_edges = edge_index.shape[1]
    align = MSEG * NS * L
    n_pad = ((n + align - 1) // align) * align

    zp = jnp.power(Z, PZBL)[atom_types]
    zq = (Z * (QQ ** 0.5))[atom_types]
    table = jnp.concatenate(
        [pos.astype(jnp.float32), zp[:, None], zq[:, None],
         jnp.zeros((n, 3), jnp.float32)], axis=1)
    edges = edge_index.astype(jnp.int32).reshape(-1)

    partial = _make_sc_kernel(n_pad, n_edges)(table, edges)
    return (partial[:n] + partial[n_pad:n_pad + n])[:, None]


# speedup vs baseline: 120.6371x; 1.3442x over previous
import functools

import jax
import jax.numpy as jnp
from jax import lax
from jax.experimental import pallas as pl
from jax.experimental.pallas import tpu as pltpu
from jax.experimental.pallas import tpu_sc as plsc

R_MAX = 5.0
QQ = 14.399645 * 0.5
PZBL = 0.23
A0 = 0.4685
C1, C2, C3, C4 = 0.02817, 0.28022, 0.50986, 0.18175
D1, D2, D3, D4 = -0.20162, -0.4029, -0.94229, -3.1998

NC = 2
NS = 16
NW = NC * NS
L = 16

CHUNK = 512
IPD = 128
MSEG = 16
NDESC = CHUNK // IPD
PIPE_AHEAD = 2


def _rsqrt(r2):
    bits = plsc.bitcast(r2, jnp.int32)
    y = plsc.bitcast(jnp.int32(0x5F3759DF) - (bits >> 1), jnp.float32)
    for _ in range(2):
        y = y * (1.5 - 0.5 * r2 * y * y)
    return y


def _edge_energy(sx, sy, sz, szp, szq, dx, dy, dz, dzp, dzq):
    ex = sx - dx
    ey = sy - dy
    ez = sz - dz
    r2 = ex * ex + ey * ey + ez * ez
    rinv = _rsqrt(r2)
    r = r2 * rinv
    x = (szp + dzp) * (r * (1.0 / A0))
    psi = (C1 * jnp.exp(D1 * x) + C2 * jnp.exp(D2 * x)
           + C3 * jnp.exp(D3 * x) + C4 * jnp.exp(D4 * x))
    eng = (szq * dzq) * rinv * psi
    rn = r * (1.0 / R_MAX)
    rn2 = rn * rn
    rn4 = rn2 * rn2
    rn6 = rn4 * rn2
    rn7 = rn6 * rn
    rn8 = rn7 * rn
    cut = 1.0 - 28.0 * rn6 + 48.0 * rn7 - 21.0 * rn8
    return jnp.where(rn < 1.0, cut * eng, 0.0)


def _make_sc_kernel(n_pad, n_edges_pad, chunks_per_worker):
    seg_len = n_pad // MSEG
    sl = seg_len // NS
    vecs = CHUNK // L

    mesh = plsc.VectorSubcoreMesh(
        core_axis_name="c", subcore_axis_name="s", num_cores=NC,
        num_subcores=NS)

    @functools.partial(
        pl.kernel,
        out_type=jax.ShapeDtypeStruct((NC * n_pad,), jnp.float32),
        mesh=mesh,
        scratch_types=[
            pltpu.VMEM((n_pad,), jnp.float32),
            pltpu.VMEM((2 * 2 * CHUNK,), jnp.int32),
            pltpu.VMEM((2 * CHUNK, 8), jnp.float32),
            pltpu.VMEM((2 * CHUNK, 8), jnp.float32),
            pltpu.VMEM((sl,), jnp.float32),
            pltpu.VMEM((sl,), jnp.float32),
            pltpu.VMEM_SHARED((NS * seg_len,), jnp.float32),
            pltpu.SemaphoreType.DMA,
            pltpu.SemaphoreType.DMA,
            pltpu.SemaphoreType.DMA,
            pltpu.SemaphoreType.DMA,
        ],
        compiler_params=pltpu.CompilerParams(
            needs_layout_passes=False, use_tc_tiling_on_sc=False),
    )
    def zbl(table_hbm, edges_hbm, out_hbm, acc, eidx, srows, drows,
            msum, mtmp, shared, si0, si1, sr0, sr1):
        cid = lax.axis_index("c")
        sid = lax.axis_index("s")
        wid = cid * NS + sid
        lane = lax.iota(jnp.int32, 16)
        zero_f = (lane * 0).astype(jnp.float32)
        sis = (si0, si1)
        srs = (sr0, sr1)

        def idx_copies(k, b):
            base = (wid + k * NW) * CHUNK
            off = b * 2 * CHUNK
            return (
                pltpu.make_async_copy(edges_hbm.at[pl.ds(base, CHUNK)],
                                      eidx.at[pl.ds(off, CHUNK)], sis[b]),
                pltpu.make_async_copy(
                    edges_hbm.at[pl.ds(n_edges_pad + base, CHUNK)],
                    eidx.at[pl.ds(off + CHUNK, CHUNK)], sis[b]),
            )

        def row_copies(b):
            off = b * 2 * CHUNK
            cps = []
            for j in range(NDESC):
                cps.append(pltpu.make_async_copy(
                    table_hbm.at[eidx.at[pl.ds(off + j * IPD, IPD)]],
                    srows.at[pl.ds(b * CHUNK + j * IPD, IPD)], srs[b]))
                cps.append(pltpu.make_async_copy(
                    table_hbm.at[eidx.at[pl.ds(off + CHUNK + j * IPD, IPD)]],
                    drows.at[pl.ds(b * CHUNK + j * IPD, IPD)], srs[b]))
            return cps

        def zero_body(i, _):
            acc[pl.ds(i * L, L)] = zero_f
            return 0

        lax.fori_loop(0, n_pad // L, zero_body, 0)

        for cp in idx_copies(0, 0):
            cp.start()
        for cp in idx_copies(1, 1):
            cp.start()
        for cp in idx_copies(0, 0):
            cp.wait()
        for cp in row_copies(0):
            cp.start()

        def compute_chunk(b):
            off = b * 2 * CHUNK

            def vec_body(v, _):
                rid = lane + v * L + b * CHUNK
                col = lane * 0
                sxv = plsc.load_gather(srows, [rid, col])
                syv = plsc.load_gather(srows, [rid, col + 1])
                szv = plsc.load_gather(srows, [rid, col + 2])
                szp = plsc.load_gather(srows, [rid, col + 3])
                szq = plsc.load_gather(srows, [rid, col + 4])
                dxv = plsc.load_gather(drows, [rid, col])
                dyv = plsc.load_gather(drows, [rid, col + 1])
                dzv = plsc.load_gather(drows, [rid, col + 2])
                dzp = plsc.load_gather(drows, [rid, col + 3])
                dzq = plsc.load_gather(drows, [rid, col + 4])
                e = _edge_energy(sxv, syv, szv, szp, szq,
                                 dxv, dyv, dzv, dzp, dzq)
                sidx = eidx[pl.ds(off + v * L, L)]
                plsc.addupdate_scatter(acc, [sidx], e)
                return 0

            lax.fori_loop(0, vecs, vec_body, 0)

        def stage(k, b):
            for cp in row_copies(b):
                cp.wait()
            for cp in idx_copies(k + 1, 1 - b):
                cp.wait()
            for cp in row_copies(1 - b):
                cp.start()
            compute_chunk(b)
            for cp in idx_copies(k + 2, b):
                cp.start()

        def pair_body(p, _):
            stage(2 * p, 0)
            stage(2 * p + 1, 1)
            return 0

        lax.fori_loop(0, chunks_per_worker // 2, pair_body, 0)

        for cp in row_copies(0):
            cp.wait()
        for cp in idx_copies(chunks_per_worker + 1, 1):
            cp.wait()

        mbase = sid * sl
        for g in range(MSEG):
            seg0 = g * seg_len
            pltpu.sync_copy(acc.at[pl.ds(seg0, seg_len)],
                            shared.at[pl.ds(sid * seg_len, seg_len)])
            plsc.subcore_barrier()
            pltpu.sync_copy(shared.at[pl.ds(mbase, sl)], msum)

            def merge_tile(t, _):
                pltpu.sync_copy(shared.at[pl.ds(t * seg_len + mbase, sl)],
                                mtmp)

                def add_body(k, _):
                    msum[pl.ds(k * L, L)] = msum[pl.ds(k * L, L)] + \
                        mtmp[pl.ds(k * L, L)]
                    return 0

                lax.fori_loop(0, sl // L, add_body, 0)
                return 0

            lax.fori_loop(1, NS, merge_tile, 0)
            pltpu.sync_copy(
                msum, out_hbm.at[pl.ds(cid * n_pad + seg0 + mbase, sl)])
            plsc.subcore_barrier()

    return zbl


def kernel(pos, Z, atom_types, edge_index):
    n = pos.shape[0]
    n_edges = edge_index.shape[1]
    align = MSEG * NS * L
    n_pad = ((n + align - 1) // align) * align

    n_chunks = -(-n_edges // CHUNK)
    cpw = -(-n_chunks // NW)
    cpw += cpw % 2
    n_edges_pad = (cpw + PIPE_AHEAD) * NW * CHUNK

    zp = jnp.power(Z, PZBL)[atom_types]
    zq = (Z * (QQ ** 0.5))[atom_types]
    table = jnp.concatenate(
        [pos.astype(jnp.float32), zp[:, None], zq[:, None],
         jnp.zeros((n, 3), jnp.float32)], axis=1)
    table = jnp.concatenate(
        [table, jnp.zeros((n_pad - n, 8), jnp.float32)], axis=0)
    edges = edge_index.astype(jnp.int32)
    pad = jnp.full((2, n_edges_pad - n_edges), n, jnp.int32)
    edges = jnp.concatenate([edges, pad], axis=1).reshape(-1)

    partial = _make_sc_kernel(n_pad, n_edges_pad, cpw)(table, edges)
    return (partial[:n] + partial[n_pad:n_pad + n])[:, None]


# speedup vs baseline: 121.3475x; 1.0059x over previous
import functools

import jax
import jax.numpy as jnp
from jax import lax
from jax.experimental import pallas as pl
from jax.experimental.pallas import tpu as pltpu
from jax.experimental.pallas import tpu_sc as plsc

R_MAX = 5.0
QQ = 14.399645 * 0.5
PZBL = 0.23
A0 = 0.4685
C1, C2, C3, C4 = 0.02817, 0.28022, 0.50986, 0.18175
D1, D2, D3, D4 = -0.20162, -0.4029, -0.94229, -3.1998

NC = 2
NS = 16
NW = NC * NS
L = 16

CHUNK = 512
IPD = 128
MSEG = 16
NDESC = CHUNK // IPD
PIPE_AHEAD = 2
UNROLL = 4


def _rsqrt(r2):
    bits = plsc.bitcast(r2, jnp.int32)
    y = plsc.bitcast(jnp.int32(0x5F3759DF) - (bits >> 1), jnp.float32)
    for _ in range(2):
        y = y * (1.5 - 0.5 * r2 * y * y)
    return y


def _edge_energy(sx, sy, sz, szp, szq, dx, dy, dz, dzp, dzq):
    ex = sx - dx
    ey = sy - dy
    ez = sz - dz
    r2 = ex * ex + ey * ey + ez * ez
    rinv = _rsqrt(r2)
    r = r2 * rinv
    x = (szp + dzp) * (r * (1.0 / A0))
    psi = (C1 * jnp.exp(D1 * x) + C2 * jnp.exp(D2 * x)
           + C3 * jnp.exp(D3 * x) + C4 * jnp.exp(D4 * x))
    eng = (szq * dzq) * rinv * psi
    rn = r * (1.0 / R_MAX)
    rn2 = rn * rn
    rn4 = rn2 * rn2
    rn6 = rn4 * rn2
    rn7 = rn6 * rn
    rn8 = rn7 * rn
    cut = 1.0 - 28.0 * rn6 + 48.0 * rn7 - 21.0 * rn8
    return jnp.where(rn < 1.0, cut * eng, 0.0)


def _make_sc_kernel(n_pad, n_edges_pad, chunks_per_worker):
    seg_len = n_pad // MSEG
    sl = seg_len // NS
    vecs = CHUNK // L

    mesh = plsc.VectorSubcoreMesh(
        core_axis_name="c", subcore_axis_name="s", num_cores=NC,
        num_subcores=NS)

    @functools.partial(
        pl.kernel,
        out_type=jax.ShapeDtypeStruct((NC * n_pad,), jnp.float32),
        mesh=mesh,
        scratch_types=[
            pltpu.VMEM((n_pad,), jnp.float32),
            pltpu.VMEM((2 * 2 * CHUNK,), jnp.int32),
            pltpu.VMEM((2 * CHUNK, 8), jnp.float32),
            pltpu.VMEM((2 * CHUNK, 8), jnp.float32),
            pltpu.VMEM((sl,), jnp.float32),
            pltpu.VMEM((sl,), jnp.float32),
            pltpu.VMEM_SHARED((NS * seg_len,), jnp.float32),
            pltpu.SemaphoreType.DMA,
            pltpu.SemaphoreType.DMA,
            pltpu.SemaphoreType.DMA,
            pltpu.SemaphoreType.DMA,
        ],
        compiler_params=pltpu.CompilerParams(
            needs_layout_passes=False, use_tc_tiling_on_sc=False),
    )
    def zbl(table_hbm, edges_hbm, out_hbm, acc, eidx, srows, drows,
            msum, mtmp, shared, si0, si1, sr0, sr1):
        cid = lax.axis_index("c")
        sid = lax.axis_index("s")
        wid = cid * NS + sid
        lane = lax.iota(jnp.int32, 16)
        zero_f = (lane * 0).astype(jnp.float32)
        sis = (si0, si1)
        srs = (sr0, sr1)

        def idx_copies(k, b):
            base = (wid + k * NW) * CHUNK
            off = b * 2 * CHUNK
            return (
                pltpu.make_async_copy(edges_hbm.at[pl.ds(base, CHUNK)],
                                      eidx.at[pl.ds(off, CHUNK)], sis[b]),
                pltpu.make_async_copy(
                    edges_hbm.at[pl.ds(n_edges_pad + base, CHUNK)],
                    eidx.at[pl.ds(off + CHUNK, CHUNK)], sis[b]),
            )

        def row_copies(b):
            off = b * 2 * CHUNK
            cps = []
            for j in range(NDESC):
                cps.append(pltpu.make_async_copy(
                    table_hbm.at[eidx.at[pl.ds(off + j * IPD, IPD)]],
                    srows.at[pl.ds(b * CHUNK + j * IPD, IPD)], srs[b]))
                cps.append(pltpu.make_async_copy(
                    table_hbm.at[eidx.at[pl.ds(off + CHUNK + j * IPD, IPD)]],
                    drows.at[pl.ds(b * CHUNK + j * IPD, IPD)], srs[b]))
            return cps

        def zero_body(i, _):
            acc[pl.ds(i * L, L)] = zero_f
            return 0

        lax.fori_loop(0, n_pad // L, zero_body, 0)

        for cp in idx_copies(0, 0):
            cp.start()
        for cp in idx_copies(1, 1):
            cp.start()
        for cp in idx_copies(0, 0):
            cp.wait()
        for cp in row_copies(0):
            cp.start()

        def compute_chunk(b):
            off = b * 2 * CHUNK

            def vec_body(v, _):
                for u in range(UNROLL):
                    rid = lane + (v * UNROLL + u) * L + b * CHUNK
                    col = lane * 0
                    sxv = plsc.load_gather(srows, [rid, col])
                    syv = plsc.load_gather(srows, [rid, col + 1])
                    szv = plsc.load_gather(srows, [rid, col + 2])
                    szp = plsc.load_gather(srows, [rid, col + 3])
                    szq = plsc.load_gather(srows, [rid, col + 4])
                    dxv = plsc.load_gather(drows, [rid, col])
                    dyv = plsc.load_gather(drows, [rid, col + 1])
                    dzv = plsc.load_gather(drows, [rid, col + 2])
                    dzp = plsc.load_gather(drows, [rid, col + 3])
                    dzq = plsc.load_gather(drows, [rid, col + 4])
                    e = _edge_energy(sxv, syv, szv, szp, szq,
                                     dxv, dyv, dzv, dzp, dzq)
                    sidx = eidx[pl.ds(off + (v * UNROLL + u) * L, L)]
                    plsc.addupdate_scatter(acc, [sidx], e)
                return 0

            lax.fori_loop(0, vecs // UNROLL, vec_body, 0)

        def stage(k, b):
            for cp in row_copies(b):
                cp.wait()
            for cp in idx_copies(k + 1, 1 - b):
                cp.wait()
            for cp in row_copies(1 - b):
                cp.start()
            compute_chunk(b)
            for cp in idx_copies(k + 2, b):
                cp.start()

        def pair_body(p, _):
            stage(2 * p, 0)
            stage(2 * p + 1, 1)
            return 0

        lax.fori_loop(0, chunks_per_worker // 2, pair_body, 0)

        for cp in row_copies(0):
            cp.wait()
        for cp in idx_copies(chunks_per_worker + 1, 1):
            cp.wait()

        mbase = sid * sl
        for g in range(MSEG):
            seg0 = g * seg_len
            pltpu.sync_copy(acc.at[pl.ds(seg0, seg_len)],
                            shared.at[pl.ds(sid * seg_len, seg_len)])
            plsc.subcore_barrier()
            pltpu.sync_copy(shared.at[pl.ds(mbase, sl)], msum)

            def merge_tile(t, _):
                pltpu.sync_copy(shared.at[pl.ds(t * seg_len + mbase, sl)],
                                mtmp)

                def add_body(k, _):
                    msum[pl.ds(k * L, L)] = msum[pl.ds(k * L, L)] + \
                        mtmp[pl.ds(k * L, L)]
                    return 0

                lax.fori_loop(0, sl // L, add_body, 0)
                return 0

            lax.fori_loop(1, NS, merge_tile, 0)
            pltpu.sync_copy(
                msum, out_hbm.at[pl.ds(cid * n_pad + seg0 + mbase, sl)])
            plsc.subcore_barrier()

    return zbl


def kernel(pos, Z, atom_types, edge_index):
    n = pos.shape[0]
    n_edges = edge_index.shape[1]
    align = MSEG * NS * L
    n_pad = ((n + align - 1) // align) * align

    n_chunks = -(-n_edges // CHUNK)
    cpw = -(-n_chunks // NW)
    cpw += cpw % 2
    n_edges_pad = (cpw + PIPE_AHEAD) * NW * CHUNK

    zp = jnp.power(Z, PZBL)[atom_types]
    zq = (Z * (QQ ** 0.5))[atom_types]
    table = jnp.concatenate(
        [pos.astype(jnp.float32), zp[:, None], zq[:, None],
         jnp.zeros((n, 3), jnp.float32)], axis=1)
    table = jnp.concatenate(
        [table, jnp.zeros((n_pad - n, 8), jnp.float32)], axis=0)
    edges = edge_index.astype(jnp.int32)
    pad = jnp.full((2, n_edges_pad - n_edges), n, jnp.int32)
    edges = jnp.concatenate([edges, pad], axis=1).reshape(-1)

    partial = _make_sc_kernel(n_pad, n_edges_pad, cpw)(table, edges)
    return (partial[:n] + partial[n_pad:n_pad + n])[:, None]


# speedup vs baseline: 121.6363x; 1.0024x over previous
import functools

import jax
import jax.numpy as jnp
from jax import lax
from jax.experimental import pallas as pl
from jax.experimental.pallas import tpu as pltpu
from jax.experimental.pallas import tpu_sc as plsc

R_MAX = 5.0
QQ = 14.399645 * 0.5
PZBL = 0.23
A0 = 0.4685
C1, C2, C3, C4 = 0.02817, 0.28022, 0.50986, 0.18175
D1, D2, D3, D4 = -0.20162, -0.4029, -0.94229, -3.1998

NC = 2
NS = 16
NW = NC * NS
L = 16

CHUNK = 512
IPD = 512
MSEG = 16
NDESC = CHUNK // IPD
PIPE_AHEAD = 2
UNROLL = 4


def _rsqrt(r2):
    bits = plsc.bitcast(r2, jnp.int32)
    y = plsc.bitcast(jnp.int32(0x5F3759DF) - (bits >> 1), jnp.float32)
    for _ in range(2):
        y = y * (1.5 - 0.5 * r2 * y * y)
    return y


def _edge_energy(sx, sy, sz, szp, szq, dx, dy, dz, dzp, dzq):
    ex = sx - dx
    ey = sy - dy
    ez = sz - dz
    r2 = ex * ex + ey * ey + ez * ez
    rinv = _rsqrt(r2)
    r = r2 * rinv
    x = (szp + dzp) * (r * (1.0 / A0))
    psi = (C1 * jnp.exp(D1 * x) + C2 * jnp.exp(D2 * x)
           + C3 * jnp.exp(D3 * x) + C4 * jnp.exp(D4 * x))
    eng = (szq * dzq) * rinv * psi
    rn = r * (1.0 / R_MAX)
    rn2 = rn * rn
    rn4 = rn2 * rn2
    rn6 = rn4 * rn2
    rn7 = rn6 * rn
    rn8 = rn7 * rn
    cut = 1.0 - 28.0 * rn6 + 48.0 * rn7 - 21.0 * rn8
    return jnp.where(rn < 1.0, cut * eng, 0.0)


def _make_sc_kernel(n_pad, n_edges_pad, chunks_per_worker):
    seg_len = n_pad // MSEG
    sl = seg_len // NS
    vecs = CHUNK // L

    mesh = plsc.VectorSubcoreMesh(
        core_axis_name="c", subcore_axis_name="s", num_cores=NC,
        num_subcores=NS)

    @functools.partial(
        pl.kernel,
        out_type=jax.ShapeDtypeStruct((NC * n_pad,), jnp.float32),
        mesh=mesh,
        scratch_types=[
            pltpu.VMEM((n_pad,), jnp.float32),
            pltpu.VMEM((2 * 2 * CHUNK,), jnp.int32),
            pltpu.VMEM((2 * CHUNK, 8), jnp.float32),
            pltpu.VMEM((2 * CHUNK, 8), jnp.float32),
            pltpu.VMEM((sl,), jnp.float32),
            pltpu.VMEM((sl,), jnp.float32),
            pltpu.VMEM_SHARED((NS * seg_len,), jnp.float32),
            pltpu.SemaphoreType.DMA,
            pltpu.SemaphoreType.DMA,
            pltpu.SemaphoreType.DMA,
            pltpu.SemaphoreType.DMA,
        ],
        compiler_params=pltpu.CompilerParams(
            needs_layout_passes=False, use_tc_tiling_on_sc=False),
    )
    def zbl(table_hbm, edges_hbm, out_hbm, acc, eidx, srows, drows,
            msum, mtmp, shared, si0, si1, sr0, sr1):
        cid = lax.axis_index("c")
        sid = lax.axis_index("s")
        wid = cid * NS + sid
        lane = lax.iota(jnp.int32, 16)
        zero_f = (lane * 0).astype(jnp.float32)
        sis = (si0, si1)
        srs = (sr0, sr1)

        def idx_copies(k, b):
            base = (wid + k * NW) * CHUNK
            off = b * 2 * CHUNK
            return (
                pltpu.make_async_copy(edges_hbm.at[pl.ds(base, CHUNK)],
                                      eidx.at[pl.ds(off, CHUNK)], sis[b]),
                pltpu.make_async_copy(
                    edges_hbm.at[pl.ds(n_edges_pad + base, CHUNK)],
                    eidx.at[pl.ds(off + CHUNK, CHUNK)], sis[b]),
            )

        def row_copies(b):
            off = b * 2 * CHUNK
            cps = []
            for j in range(NDESC):
                cps.append(pltpu.make_async_copy(
                    table_hbm.at[eidx.at[pl.ds(off + j * IPD, IPD)]],
                    srows.at[pl.ds(b * CHUNK + j * IPD, IPD)], srs[b]))
                cps.append(pltpu.make_async_copy(
                    table_hbm.at[eidx.at[pl.ds(off + CHUNK + j * IPD, IPD)]],
                    drows.at[pl.ds(b * CHUNK + j * IPD, IPD)], srs[b]))
            return cps

        def zero_body(i, _):
            acc[pl.ds(i * L, L)] = zero_f
            return 0

        lax.fori_loop(0, n_pad // L, zero_body, 0)

        for cp in idx_copies(0, 0):
            cp.start()
        for cp in idx_copies(1, 1):
            cp.start()
        for cp in idx_copies(0, 0):
            cp.wait()
        for cp in row_copies(0):
            cp.start()

        def compute_chunk(b):
            off = b * 2 * CHUNK

            def vec_body(v, _):
                for u in range(UNROLL):
                    rid = lane + (v * UNROLL + u) * L + b * CHUNK
                    col = lane * 0
                    sxv = plsc.load_gather(srows, [rid, col])
                    syv = plsc.load_gather(srows, [rid, col + 1])
                    szv = plsc.load_gather(srows, [rid, col + 2])
                    szp = plsc.load_gather(srows, [rid, col + 3])
                    szq = plsc.load_gather(srows, [rid, col + 4])
                    dxv = plsc.load_gather(drows, [rid, col])
                    dyv = plsc.load_gather(drows, [rid, col + 1])
                    dzv = plsc.load_gather(drows, [rid, col + 2])
                    dzp = plsc.load_gather(drows, [rid, col + 3])
                    dzq = plsc.load_gather(drows, [rid, col + 4])
                    e = _edge_energy(sxv, syv, szv, szp, szq,
                                     dxv, dyv, dzv, dzp, dzq)
                    sidx = eidx[pl.ds(off + (v * UNROLL + u) * L, L)]
                    plsc.addupdate_scatter(acc, [sidx], e)
                return 0

            lax.fori_loop(0, vecs // UNROLL, vec_body, 0)

        def stage(k, b):
            for cp in row_copies(b):
                cp.wait()
            for cp in idx_copies(k + 1, 1 - b):
                cp.wait()
            for cp in row_copies(1 - b):
                cp.start()
            compute_chunk(b)
            for cp in idx_copies(k + 2, b):
                cp.start()

        def pair_body(p, _):
            stage(2 * p, 0)
            stage(2 * p + 1, 1)
            return 0

        lax.fori_loop(0, chunks_per_worker // 2, pair_body, 0)

        for cp in row_copies(0):
            cp.wait()
        for cp in idx_copies(chunks_per_worker + 1, 1):
            cp.wait()

        mbase = sid * sl
        for g in range(MSEG):
            seg0 = g * seg_len
            pltpu.sync_copy(acc.at[pl.ds(seg0, seg_len)],
                            shared.at[pl.ds(sid * seg_len, seg_len)])
            plsc.subcore_barrier()
            pltpu.sync_copy(shared.at[pl.ds(mbase, sl)], msum)

            def merge_tile(t, _):
                pltpu.sync_copy(shared.at[pl.ds(t * seg_len + mbase, sl)],
                                mtmp)

                def add_body(k, _):
                    msum[pl.ds(k * L, L)] = msum[pl.ds(k * L, L)] + \
                        mtmp[pl.ds(k * L, L)]
                    return 0

                lax.fori_loop(0, sl // L, add_body, 0)
                return 0

            lax.fori_loop(1, NS, merge_tile, 0)
            pltpu.sync_copy(
                msum, out_hbm.at[pl.ds(cid * n_pad + seg0 + mbase, sl)])
            plsc.subcore_barrier()

    return zbl


def kernel(pos, Z, atom_types, edge_index):
    n = pos.shape[0]
    n_edges = edge_index.shape[1]
    align = MSEG * NS * L
    n_pad = ((n + align - 1) // align) * align

    n_chunks = -(-n_edges // CHUNK)
    cpw = -(-n_chunks // NW)
    cpw += cpw % 2
    n_edges_pad = (cpw + PIPE_AHEAD) * NW * CHUNK

    zp = jnp.power(Z, PZBL)[atom_types]
    zq = (Z * (QQ ** 0.5))[atom_types]
    table = jnp.concatenate(
        [pos.astype(jnp.float32), zp[:, None], zq[:, None],
         jnp.zeros((n, 3), jnp.float32)], axis=1)
    table = jnp.concatenate(
        [table, jnp.zeros((n_pad - n, 8), jnp.float32)], axis=0)
    edges = edge_index.astype(jnp.int32)
    pad = jnp.full((2, n_edges_pad - n_edges), n, jnp.int32)
    edges = jnp.concatenate([edges, pad], axis=1).reshape(-1)

    partial = _make_sc_kernel(n_pad, n_edges_pad, cpw)(table, edges)
    return (partial[:n] + partial[n_pad:n_pad + n])[:, None]


# speedup vs baseline: 182.7809x; 1.5027x over previous
import functools

import jax
import jax.numpy as jnp
from jax import lax
from jax.experimental import pallas as pl
from jax.experimental.pallas import tpu as pltpu
from jax.experimental.pallas import tpu_sc as plsc

R_MAX = 5.0
QQ = 14.399645 * 0.5
PZBL = 0.23
A0 = 0.4685
C1, C2, C3, C4 = 0.02817, 0.28022, 0.50986, 0.18175
D1, D2, D3, D4 = -0.20162, -0.4029, -0.94229, -3.1998

NC = 2
NS = 16
NW = NC * NS
L = 16

CHUNK = 1024
IPD = 512
NDESC = CHUNK // IPD
PIPE_AHEAD = 2
UNROLL = 4


def _rsqrt(r2):
    bits = plsc.bitcast(r2, jnp.int32)
    y = plsc.bitcast(jnp.int32(0x5F3759DF) - (bits >> 1), jnp.float32)
    for _ in range(2):
        y = y * (1.5 - 0.5 * r2 * y * y)
    return y


def _edge_energy(sx, sy, sz, szp, szq, dx, dy, dz, dzp, dzq):
    ex = sx - dx
    ey = sy - dy
    ez = sz - dz
    r2 = ex * ex + ey * ey + ez * ez
    rinv = _rsqrt(r2)
    r = r2 * rinv
    x = (szp + dzp) * (r * (1.0 / A0))
    psi = (C1 * jnp.exp(D1 * x) + C2 * jnp.exp(D2 * x)
           + C3 * jnp.exp(D3 * x) + C4 * jnp.exp(D4 * x))
    eng = (szq * dzq) * rinv * psi
    rn = r * (1.0 / R_MAX)
    rn2 = rn * rn
    rn4 = rn2 * rn2
    rn6 = rn4 * rn2
    rn7 = rn6 * rn
    rn8 = rn7 * rn
    cut = 1.0 - 28.0 * rn6 + 48.0 * rn7 - 21.0 * rn8
    return jnp.where(rn < 1.0, cut * eng, 0.0)


def _make_sc_kernel(n_pad, n_edges_pad, chunks_per_worker):
    sl = n_pad // NS
    trows = n_pad // NS
    vecs = CHUNK // L

    mesh = plsc.VectorSubcoreMesh(
        core_axis_name="c", subcore_axis_name="s", num_cores=NC,
        num_subcores=NS)

    buf_i32 = pltpu.VMEM((CHUNK,), jnp.int32)
    buf_rows = pltpu.VMEM((CHUNK, 8), jnp.float32)
    buf_f32 = pltpu.VMEM((CHUNK,), jnp.float32)
    buf_half = pltpu.VMEM((IPD,), jnp.int32)

    @functools.partial(
        pl.kernel,
        out_type=jax.ShapeDtypeStruct((NC * n_pad,), jnp.float32),
        mesh=mesh,
        scratch_types=[
            buf_i32, buf_i32,
            buf_i32, buf_i32,
            buf_rows, buf_rows,
            buf_rows, buf_rows,
            buf_f32, buf_f32,
            buf_half, buf_half,
            buf_half, buf_half,
            pltpu.VMEM((n_pad // NS,), jnp.float32),
            pltpu.VMEM_SHARED((n_pad, 8), jnp.float32),
            pltpu.VMEM_SHARED((n_pad,), jnp.float32),
            pltpu.SemaphoreType.DMA,
            pltpu.SemaphoreType.DMA,
            pltpu.SemaphoreType.DMA,
            pltpu.SemaphoreType.DMA,
            pltpu.SemaphoreType.DMA,
            pltpu.SemaphoreType.DMA,
            pltpu.SemaphoreType.DMA,
        ],
        compiler_params=pltpu.CompilerParams(
            needs_layout_passes=False, use_tc_tiling_on_sc=False),
    )
    def zbl(table_hbm, edges_hbm, out_hbm,
            esrc0, esrc1, edst0, edst1, srows0, srows1, drows0, drows1,
            ener0, ener1, ssrc0a, ssrc0b, ssrc1a, ssrc1b, zslice,
            table_sh, acc_sh, si0, si1, sr0, sr1, sc0, sc1, st):
        cid = lax.axis_index("c")
        sid = lax.axis_index("s")
        wid = cid * NS + sid
        lane = lax.iota(jnp.int32, 16)
        zero_f = (lane * 0).astype(jnp.float32)
        esrc = (esrc0, esrc1)
        edst = (edst0, edst1)
        srows = (srows0, srows1)
        drows = (drows0, drows1)
        ener = (ener0, ener1)
        ssrc = ((ssrc0a, ssrc0b), (ssrc1a, ssrc1b))
        sis = (si0, si1)
        srs = (sr0, sr1)
        scs = (sc0, sc1)

        def idx_copies(k, b):
            base = (wid + k * NW) * CHUNK
            return (
                pltpu.make_async_copy(edges_hbm.at[pl.ds(base, CHUNK)],
                                      esrc[b], sis[b]),
                pltpu.make_async_copy(
                    edges_hbm.at[pl.ds(n_edges_pad + base, CHUNK)],
                    edst[b], sis[b]),
            )

        def row_copies(b):
            cps = []
            for j in range(NDESC):
                cps.append(pltpu.make_async_copy(
                    table_sh.at[esrc[b].at[pl.ds(j * IPD, IPD)]],
                    srows[b].at[pl.ds(j * IPD, IPD)], srs[b]))
                cps.append(pltpu.make_async_copy(
                    table_sh.at[edst[b].at[pl.ds(j * IPD, IPD)]],
                    drows[b].at[pl.ds(j * IPD, IPD)], srs[b]))
            return cps

        def scat_copies(b):
            return [
                pltpu.make_async_copy(ener[b].at[pl.ds(h * IPD, IPD)],
                                      acc_sh.at[ssrc[b][h]], scs[b])
                for h in range(NDESC)
            ]

        for cp in idx_copies(0, 0):
            cp.start()
        for cp in idx_copies(1, 1):
            cp.start()
        tcp = pltpu.make_async_copy(
            table_hbm.at[pl.ds(sid * trows, trows)],
            table_sh.at[pl.ds(sid * trows, trows)], st)
        tcp.start()

        def zero_body(i, _):
            zslice[pl.ds(i * L, L)] = zero_f
            return 0

        lax.fori_loop(0, sl // L, zero_body, 0)
        pltpu.sync_copy(zslice, acc_sh.at[pl.ds(sid * sl, sl)])
        tcp.wait()
        plsc.subcore_barrier()

        for cp in idx_copies(0, 0):
            cp.wait()
        for cp in row_copies(0):
            cp.start()

        def compute_chunk(b):
            def vec_body(v, _):
                for u in range(UNROLL):
                    g = v * UNROLL + u
                    rid = lane + g * L
                    col = lane * 0
                    sxv = plsc.load_gather(srows[b], [rid, col])
                    syv = plsc.load_gather(srows[b], [rid, col + 1])
                    szv = plsc.load_gather(srows[b], [rid, col + 2])
                    szp = plsc.load_gather(srows[b], [rid, col + 3])
                    szq = plsc.load_gather(srows[b], [rid, col + 4])
                    dxv = plsc.load_gather(drows[b], [rid, col])
                    dyv = plsc.load_gather(drows[b], [rid, col + 1])
                    dzv = plsc.load_gather(drows[b], [rid, col + 2])
                    dzp = plsc.load_gather(drows[b], [rid, col + 3])
                    dzq = plsc.load_gather(drows[b], [rid, col + 4])
                    e = _edge_energy(sxv, syv, szv, szp, szq,
                                     dxv, dyv, dzv, dzp, dzq)
                    ener[b][pl.ds(g * L, L)] = e
                return 0

            lax.fori_loop(0, vecs // UNROLL, vec_body, 0)

        def stage(k, b, first):
            for cp in row_copies(b):
                cp.wait()
            for cp in idx_copies(k + 1, 1 - b):
                cp.wait()
            for cp in row_copies(1 - b):
                cp.start()
            if not first:
                for cp in scat_copies(b):
                    cp.wait()
            compute_chunk(b)
            def snap_body(i, _):
                for h in range(NDESC):
                    ssrc[b][h][pl.ds(i * L, L)] = \
                        esrc[b][pl.ds(h * IPD + i * L, L)]
                return 0

            lax.fori_loop(0, IPD // L, snap_body, 0)
            for h in range(NDESC):
                pltpu.async_copy(ener[b].at[pl.ds(h * IPD, IPD)],
                                 acc_sh.at[ssrc[b][h]], scs[b], add=True)
            for cp in idx_copies(k + 2, b):
                cp.start()

        stage(0, 0, True)
        stage(1, 1, True)

        def pair_body(p, _):
            stage(2 * p, 0, False)
            stage(2 * p + 1, 1, False)
            return 0

        lax.fori_loop(1, chunks_per_worker // 2, pair_body, 0)

        for cp in row_copies(0):
            cp.wait()
        for cp in idx_copies(chunks_per_worker + 1, 1):
            cp.wait()
        for cp in scat_copies(0):
            cp.wait()
        for cp in scat_copies(1):
            cp.wait()
        plsc.subcore_barrier()

        pltpu.sync_copy(acc_sh.at[pl.ds(sid * sl, sl)], zslice)
        pltpu.sync_copy(zslice, out_hbm.at[pl.ds(cid * n_pad + sid * sl, sl)])

    return zbl


def kernel(pos, Z, atom_types, edge_index):
    n = pos.shape[0]
    n_edges = edge_index.shape[1]
    align = NS * L
    n_pad = ((n + align - 1) // align) * align

    n_chunks = -(-n_edges // CHUNK)
    cpw = -(-n_chunks // NW)
    cpw += cpw % 2
    n_edges_pad = (cpw + PIPE_AHEAD) * NW * CHUNK

    zp = jnp.power(Z, PZBL)[atom_types]
    zq = (Z * (QQ ** 0.5))[atom_types]
    table = jnp.concatenate(
        [pos.astype(jnp.float32), zp[:, None], zq[:, None],
         jnp.zeros((n, 3), jnp.float32)], axis=1)
    table = jnp.concatenate(
        [table, jnp.zeros((n_pad - n, 8), jnp.float32)], axis=0)
    edges = edge_index.astype(jnp.int32)
    pad = jnp.full((2, n_edges_pad - n_edges), n, jnp.int32)
    edges = jnp.concatenate([edges, pad], axis=1).reshape(-1)

    partial = _make_sc_kernel(n_pad, n_edges_pad, cpw)(table, edges)
    return (partial[:n] + partial[n_pad:n_pad + n])[:, None]
